# Initial kernel scaffold; baseline (speedup 1.0000x reference)
import jax
import jax.numpy as jnp
from jax.experimental import pallas as pl


def kernel(feat_p, feat_s, edge_index_p, edge_index_s, W1, b1, W2, b2, W3, b3, Wr, br, Wm1, bm1, Wm2, bm2):
    raise NotImplementedError("write your pallas kernel here")


# speedup vs baseline: 60.5468x; 60.5468x over previous
import functools

import jax
import jax.numpy as jnp
from jax import lax
from jax.experimental import pallas as pl
from jax.experimental.pallas import tpu as pltpu
from jax.experimental.pallas import tpu_sc as plsc

N = 10000
E = 320000
D = 128
NROW = 80
NPAD = NROW * 128
NTILES = 16
EC = E // NTILES
VEC = EC // 16
CHUNK = NROW // NTILES


def _zero(ref):
    def body(r, _):
        for j in range(8):
            ref[r, pl.ds(j * 16, 16)] = jnp.zeros((16,), jnp.float32)
        return _
    lax.fori_loop(0, NROW, body, None)


def _sc_body(src_hbm, dst_hbm, u3_out, aux_out,
             src_v, dst_v, w_v, inv_v, acc_v, tmp_v, idx_v, aux_v,
             sh_cnt, sh_u1, sh_u2, sh_u3):
    c = lax.axis_index("c")
    s = lax.axis_index("s")
    base = c * E + s * EC

    pltpu.sync_copy(src_hbm.at[pl.ds(base, EC)], src_v)
    pltpu.sync_copy(dst_hbm.at[pl.ds(base, EC)], dst_v)

    iota = lax.iota(jnp.int32, 16)
    for i in range(NROW // 16):
        idx_v[pl.ds(i * 16, 16)] = iota + i * 16

    _zero(acc_v)

    for tid, sh in ((0, sh_cnt), (1, sh_u1), (2, sh_u2), (3, sh_u3)):
        @pl.when(s == tid)
        def _():
            pltpu.sync_copy(acc_v, sh)
    plsc.subcore_barrier()

    ones = jnp.ones((16,), jnp.float32)

    def cnt_body(i, _):
        dv = dst_v[pl.ds(i * 16, 16)]
        plsc.addupdate_scatter(acc_v, [lax.shift_right_logical(dv, 7), dv & 127], ones)
        return _
    lax.fori_loop(0, VEC, cnt_body, None)
    pltpu.sync_copy(acc_v, sh_cnt.at[idx_v], add=True)
    plsc.subcore_barrier()

    pltpu.sync_copy(sh_cnt, tmp_v)

    def inv_body(r, _):
        for j in range(8):
            sl = pl.ds(j * 16, 16)
            inv_v[r, sl] = 1.0 / jnp.maximum(tmp_v[r, sl], 1.0)
        return _
    lax.fori_loop(0, NROW, inv_body, None)

    def edge_pass(w_ref, sh_dst):
        _zero(acc_v)

        def body(i, _):
            sl = pl.ds(i * 16, 16)
            dv = dst_v[sl]
            sv = src_v[sl]
            vals = plsc.load_gather(w_ref, [lax.shift_right_logical(dv, 7), dv & 127])
            plsc.addupdate_scatter(acc_v, [lax.shift_right_logical(sv, 7), sv & 127], vals)
            return _
        lax.fori_loop(0, VEC, body, None)
        pltpu.sync_copy(acc_v, sh_dst.at[idx_v], add=True)
        plsc.subcore_barrier()

    def finish_pass(sh_src, last):
        pltpu.sync_copy(sh_src, tmp_v)

        def body(r, acc):
            for j in range(8):
                sl = pl.ds(j * 16, 16)
                uv = tmp_v[r, sl]
                if not last:
                    w_v[r, sl] = uv * inv_v[r, sl]
                acc = acc + uv
            return acc
        acc = lax.fori_loop(0, NROW, body, jnp.zeros((16,), jnp.float32))
        return lax.reduce_sum_p.bind(acc, axes=(0,))

    edge_pass(inv_v, sh_u1)
    s1 = finish_pass(sh_u1, last=False)
    edge_pass(w_v, sh_u2)
    s2 = finish_pass(sh_u2, last=False)
    edge_pass(w_v, sh_u3)

    @pl.when(s < NROW // 8)
    def _():
        pltpu.sync_copy(sh_u3.at[pl.ds(s * 8, 8)],
                        u3_out.at[c, pl.ds(s * 8, 8)])

    @pl.when(s == 0)
    def _():
        lane = lax.iota(jnp.int32, 16)
        vec = jnp.where(lane == 0, jnp.full((16,), s1, jnp.float32),
                        jnp.where(lane == 1, jnp.full((16,), s2, jnp.float32),
                                  jnp.zeros((16,), jnp.float32)))
        aux_v[...] = vec
        pltpu.sync_copy(aux_v, aux_out.at[pl.ds(c * 16, 16)])


def _sc_call(src2, dst2):
    mesh = plsc.VectorSubcoreMesh(core_axis_name="c", subcore_axis_name="s")
    kern = functools.partial(
        pl.kernel,
        mesh=mesh,
        compiler_params=pltpu.CompilerParams(needs_layout_passes=False),
        out_type=(jax.ShapeDtypeStruct((2, NROW, 128), jnp.float32),
                  jax.ShapeDtypeStruct((32,), jnp.float32)),
        scratch_types=[
            pltpu.VMEM((EC,), jnp.int32),
            pltpu.VMEM((EC,), jnp.int32),
            pltpu.VMEM((NROW, 128), jnp.float32),
            pltpu.VMEM((NROW, 128), jnp.float32),
            pltpu.VMEM((NROW, 128), jnp.float32),
            pltpu.VMEM((NROW, 128), jnp.float32),
            pltpu.VMEM((NROW,), jnp.int32),
            pltpu.VMEM((16,), jnp.float32),
            pltpu.VMEM_SHARED((NROW, 128), jnp.float32),
            pltpu.VMEM_SHARED((NROW, 128), jnp.float32),
            pltpu.VMEM_SHARED((NROW, 128), jnp.float32),
            pltpu.VMEM_SHARED((NROW, 128), jnp.float32),
        ],
    )(_sc_body)
    return kern(src2, dst2)


NBLK = 10
BLK = NPAD // NBLK


def _tc_body(u3_ref, fp_ref, fs_ref, aux_ref,
             W1_ref, b1_ref, W2_ref, b2_ref, W3_ref, b3_ref,
             Wr_ref, br_ref, Wm1_ref, bm1_ref, Wm2_ref, bm2_ref,
             out_ref, y_acc):
    i = pl.program_id(0)

    @pl.when(i == 0)
    def _():
        y_acc[...] = jnp.zeros_like(y_acc)

    u3 = u3_ref[...]
    yp = jnp.dot(u3[0:1, :], fp_ref[...], preferred_element_type=jnp.float32)
    ys = jnp.dot(u3[1:2, :], fs_ref[...], preferred_element_type=jnp.float32)
    y_acc[...] += jnp.concatenate([yp, ys], axis=0)

    @pl.when(i == NBLK - 1)
    def _():
        y = y_acc[...]
        aux = aux_ref[...]
        s1 = aux[:, 0:1]
        s2 = aux[:, 1:2]
        b1 = b1_ref[...]
        b2 = b2_ref[...]
        b3 = b3_ref[...]
        t = jnp.dot(y, W1_ref[...].T, preferred_element_type=jnp.float32) + s2 * b1
        t = jnp.dot(t, W2_ref[...].T, preferred_element_type=jnp.float32) + s1 * b2
        t = jnp.dot(t, W3_ref[...].T, preferred_element_type=jnp.float32)
        g = t * (1.0 / N) + b3
        z = jnp.dot(g, Wr_ref[...].T, preferred_element_type=jnp.float32) + br_ref[...]
        f = 1.0 / (1.0 + jnp.exp(-z))
        cat = jnp.concatenate([f[0:1, :], f[1:2, :]], axis=1)
        d1 = jnp.dot(cat, Wm1_ref[...].T, preferred_element_type=jnp.float32) + bm1_ref[...]
        d2 = jnp.sum(d1 * Wm2_ref[...], axis=1, keepdims=True) + bm2_ref[...]
        out_ref[...] = 1.0 / (1.0 + jnp.exp(-d2))


def _tc_call(u3r, fp_pad, fs_pad, aux, W1, b1, W2, b2, W3, b3, Wr, br, Wm1, bm1, Wm2, bm2):
    full = lambda shape: pl.BlockSpec(shape, lambda i: (0,) * len(shape))
    return pl.pallas_call(
        _tc_body,
        grid=(NBLK,),
        in_specs=[
            pl.BlockSpec((2, BLK), lambda i: (0, i)),
            pl.BlockSpec((BLK, D), lambda i: (i, 0)),
            pl.BlockSpec((BLK, D), lambda i: (i, 0)),
            full((2, 16)),
            full((D, D)), full((1, D)),
            full((D, D)), full((1, D)),
            full((D, D)), full((1, D)),
            full((D, D)), full((1, D)),
            full((D, 2 * D)), full((1, D)),
            full((1, D)), full((1, 1)),
        ],
        out_specs=pl.BlockSpec((1, 1), lambda i: (0, 0)),
        out_shape=jax.ShapeDtypeStruct((1, 1), jnp.float32),
        scratch_shapes=[pltpu.VMEM((2, D), jnp.float32)],
    )(u3r, fp_pad, fs_pad, aux, W1, b1, W2, b2, W3, b3, Wr, br, Wm1, bm1, Wm2, bm2)


def kernel(feat_p, feat_s, edge_index_p, edge_index_s,
           W1, b1, W2, b2, W3, b3, Wr, br, Wm1, bm1, Wm2, bm2):
    src2 = jnp.concatenate([edge_index_p[0], edge_index_s[0]])
    dst2 = jnp.concatenate([edge_index_p[1], edge_index_s[1]])
    u3, aux = _sc_call(src2, dst2)
    aux = aux.reshape(2, 16)
    u3r = u3.reshape(2, NPAD)
    pad = ((0, NPAD - N), (0, 0))
    fp_pad = jnp.pad(feat_p, pad)
    fs_pad = jnp.pad(feat_s, pad)
    out = _tc_call(u3r, fp_pad, fs_pad, aux,
                   W1, b1.reshape(1, D), W2, b2.reshape(1, D),
                   W3, b3.reshape(1, D), Wr, br.reshape(1, D),
                   Wm1, bm1.reshape(1, D), Wm2, bm2.reshape(1, 1))
    return out.reshape(1)


# speedup vs baseline: 60.8767x; 1.0054x over previous
import functools

import jax
import jax.numpy as jnp
from jax import lax
from jax.experimental import pallas as pl
from jax.experimental.pallas import tpu as pltpu
from jax.experimental.pallas import tpu_sc as plsc

N = 10000
E = 320000
D = 128
NROW = 80
NPAD = NROW * 128
NTILES = 16
EC = E // NTILES
VEC = EC // 16
CHUNK = NROW // NTILES
UNROLL = 5


def _zero(ref):
    def body(r, _):
        for j in range(8):
            ref[r, pl.ds(j * 16, 16)] = jnp.zeros((16,), jnp.float32)
        return _
    lax.fori_loop(0, NROW, body, None)


def _sc_body(src_hbm, dst_hbm, u3_out, aux_out,
             src_v, dst_v, w_v, inv_v, acc_v, tmp_v, idx_v, aux_v,
             sh_cnt, sh_u1, sh_u2, sh_u3):
    c = lax.axis_index("c")
    s = lax.axis_index("s")
    base = c * E + s * EC

    pltpu.sync_copy(src_hbm.at[pl.ds(base, EC)], src_v)
    pltpu.sync_copy(dst_hbm.at[pl.ds(base, EC)], dst_v)

    iota = lax.iota(jnp.int32, 16)
    for i in range(NROW // 16):
        idx_v[pl.ds(i * 16, 16)] = iota + i * 16

    _zero(acc_v)

    for tid, sh in ((0, sh_cnt), (1, sh_u1), (2, sh_u2), (3, sh_u3)):
        @pl.when(s == tid)
        def _():
            pltpu.sync_copy(acc_v, sh)
    plsc.subcore_barrier()

    ones = jnp.ones((16,), jnp.float32)

    def cnt_body(i, _):
        for u in range(UNROLL):
            dv = dst_v[pl.ds(i * (16 * UNROLL) + u * 16, 16)]
            plsc.addupdate_scatter(acc_v, [lax.shift_right_logical(dv, 7), dv & 127], ones)
        return _
    lax.fori_loop(0, VEC // UNROLL, cnt_body, None)
    pltpu.sync_copy(acc_v, sh_cnt.at[idx_v], add=True)
    plsc.subcore_barrier()

    pltpu.sync_copy(sh_cnt, tmp_v)

    def inv_body(r, _):
        for j in range(8):
            sl = pl.ds(j * 16, 16)
            inv_v[r, sl] = 1.0 / jnp.maximum(tmp_v[r, sl], 1.0)
        return _
    lax.fori_loop(0, NROW, inv_body, None)

    def edge_pass(w_ref, sh_dst):
        _zero(acc_v)

        def body(i, _):
            for u in range(UNROLL):
                sl = pl.ds(i * (16 * UNROLL) + u * 16, 16)
                dv = dst_v[sl]
                sv = src_v[sl]
                vals = plsc.load_gather(w_ref, [lax.shift_right_logical(dv, 7), dv & 127])
                plsc.addupdate_scatter(acc_v, [lax.shift_right_logical(sv, 7), sv & 127], vals)
            return _
        lax.fori_loop(0, VEC // UNROLL, body, None)
        pltpu.sync_copy(acc_v, sh_dst.at[idx_v], add=True)
        plsc.subcore_barrier()

    def finish_pass(sh_src, last):
        pltpu.sync_copy(sh_src, tmp_v)

        def body(r, acc):
            for j in range(8):
                sl = pl.ds(j * 16, 16)
                uv = tmp_v[r, sl]
                if not last:
                    w_v[r, sl] = uv * inv_v[r, sl]
                acc = acc + uv
            return acc
        acc = lax.fori_loop(0, NROW, body, jnp.zeros((16,), jnp.float32))
        return lax.reduce_sum_p.bind(acc, axes=(0,))

    edge_pass(inv_v, sh_u1)
    s1 = finish_pass(sh_u1, last=False)
    edge_pass(w_v, sh_u2)
    s2 = finish_pass(sh_u2, last=False)
    edge_pass(w_v, sh_u3)

    @pl.when(s < NROW // 8)
    def _():
        pltpu.sync_copy(sh_u3.at[pl.ds(s * 8, 8)],
                        u3_out.at[c, pl.ds(s * 8, 8)])

    @pl.when(s == 0)
    def _():
        lane = lax.iota(jnp.int32, 16)
        vec = jnp.where(lane == 0, jnp.full((16,), s1, jnp.float32),
                        jnp.where(lane == 1, jnp.full((16,), s2, jnp.float32),
                                  jnp.zeros((16,), jnp.float32)))
        aux_v[...] = vec
        pltpu.sync_copy(aux_v, aux_out.at[pl.ds(c * 16, 16)])


def _sc_call(src2, dst2):
    mesh = plsc.VectorSubcoreMesh(core_axis_name="c", subcore_axis_name="s")
    kern = functools.partial(
        pl.kernel,
        mesh=mesh,
        compiler_params=pltpu.CompilerParams(needs_layout_passes=False),
        out_type=(jax.ShapeDtypeStruct((2, NROW, 128), jnp.float32),
                  jax.ShapeDtypeStruct((32,), jnp.float32)),
        scratch_types=[
            pltpu.VMEM((EC,), jnp.int32),
            pltpu.VMEM((EC,), jnp.int32),
            pltpu.VMEM((NROW, 128), jnp.float32),
            pltpu.VMEM((NROW, 128), jnp.float32),
            pltpu.VMEM((NROW, 128), jnp.float32),
            pltpu.VMEM((NROW, 128), jnp.float32),
            pltpu.VMEM((NROW,), jnp.int32),
            pltpu.VMEM((16,), jnp.float32),
            pltpu.VMEM_SHARED((NROW, 128), jnp.float32),
            pltpu.VMEM_SHARED((NROW, 128), jnp.float32),
            pltpu.VMEM_SHARED((NROW, 128), jnp.float32),
            pltpu.VMEM_SHARED((NROW, 128), jnp.float32),
        ],
    )(_sc_body)
    return kern(src2, dst2)


NBLK = 10
BLK = NPAD // NBLK


def _tc_body(u3_ref, fp_ref, fs_ref, aux_ref,
             W1_ref, b1_ref, W2_ref, b2_ref, W3_ref, b3_ref,
             Wr_ref, br_ref, Wm1_ref, bm1_ref, Wm2_ref, bm2_ref,
             out_ref, y_acc):
    i = pl.program_id(0)

    @pl.when(i == 0)
    def _():
        y_acc[...] = jnp.zeros_like(y_acc)

    u3 = u3_ref[...]
    yp = jnp.dot(u3[0:1, :], fp_ref[...], preferred_element_type=jnp.float32)
    ys = jnp.dot(u3[1:2, :], fs_ref[...], preferred_element_type=jnp.float32)
    y_acc[...] += jnp.concatenate([yp, ys], axis=0)

    @pl.when(i == NBLK - 1)
    def _():
        y = y_acc[...]
        aux = aux_ref[...]
        s1 = aux[:, 0:1]
        s2 = aux[:, 1:2]
        b1 = b1_ref[...]
        b2 = b2_ref[...]
        b3 = b3_ref[...]
        t = jnp.dot(y, W1_ref[...].T, preferred_element_type=jnp.float32) + s2 * b1
        t = jnp.dot(t, W2_ref[...].T, preferred_element_type=jnp.float32) + s1 * b2
        t = jnp.dot(t, W3_ref[...].T, preferred_element_type=jnp.float32)
        g = t * (1.0 / N) + b3
        z = jnp.dot(g, Wr_ref[...].T, preferred_element_type=jnp.float32) + br_ref[...]
        f = 1.0 / (1.0 + jnp.exp(-z))
        cat = jnp.concatenate([f[0:1, :], f[1:2, :]], axis=1)
        d1 = jnp.dot(cat, Wm1_ref[...].T, preferred_element_type=jnp.float32) + bm1_ref[...]
        d2 = jnp.sum(d1 * Wm2_ref[...], axis=1, keepdims=True) + bm2_ref[...]
        out_ref[...] = 1.0 / (1.0 + jnp.exp(-d2))


def _tc_call(u3r, fp_pad, fs_pad, aux, W1, b1, W2, b2, W3, b3, Wr, br, Wm1, bm1, Wm2, bm2):
    full = lambda shape: pl.BlockSpec(shape, lambda i: (0,) * len(shape))
    return pl.pallas_call(
        _tc_body,
        grid=(NBLK,),
        in_specs=[
            pl.BlockSpec((2, BLK), lambda i: (0, i)),
            pl.BlockSpec((BLK, D), lambda i: (i, 0)),
            pl.BlockSpec((BLK, D), lambda i: (i, 0)),
            full((2, 16)),
            full((D, D)), full((1, D)),
            full((D, D)), full((1, D)),
            full((D, D)), full((1, D)),
            full((D, D)), full((1, D)),
            full((D, 2 * D)), full((1, D)),
            full((1, D)), full((1, 1)),
        ],
        out_specs=pl.BlockSpec((1, 1), lambda i: (0, 0)),
        out_shape=jax.ShapeDtypeStruct((1, 1), jnp.float32),
        scratch_shapes=[pltpu.VMEM((2, D), jnp.float32)],
    )(u3r, fp_pad, fs_pad, aux, W1, b1, W2, b2, W3, b3, Wr, br, Wm1, bm1, Wm2, bm2)


def kernel(feat_p, feat_s, edge_index_p, edge_index_s,
           W1, b1, W2, b2, W3, b3, Wr, br, Wm1, bm1, Wm2, bm2):
    src2 = jnp.concatenate([edge_index_p[0], edge_index_s[0]])
    dst2 = jnp.concatenate([edge_index_p[1], edge_index_s[1]])
    u3, aux = _sc_call(src2, dst2)
    aux = aux.reshape(2, 16)
    u3r = u3.reshape(2, NPAD)
    pad = ((0, NPAD - N), (0, 0))
    fp_pad = jnp.pad(feat_p, pad)
    fs_pad = jnp.pad(feat_s, pad)
    out = _tc_call(u3r, fp_pad, fs_pad, aux,
                   W1, b1.reshape(1, D), W2, b2.reshape(1, D),
                   W3, b3.reshape(1, D), Wr, br.reshape(1, D),
                   Wm1, bm1.reshape(1, D), Wm2, bm2.reshape(1, 1))
    return out.reshape(1)


# speedup vs baseline: 104.3431x; 1.7140x over previous
import functools

import jax
import jax.numpy as jnp
from jax import lax
from jax.experimental import pallas as pl
from jax.experimental.pallas import tpu as pltpu
from jax.experimental.pallas import tpu_sc as plsc

N = 10000
E = 320000
D = 128
NROW = 80
NPAD = NROW * 128
NTILES = 16
EC = E // NTILES
VEC = EC // 16
CHUNK = NROW // NTILES
UNROLL = 5


def _zero(ref):
    @plsc.parallel_loop(0, NROW)
    def _(r):
        for j in range(8):
            ref[r, pl.ds(j * 16, 16)] = jnp.zeros((16,), jnp.float32)


def _sc_body(ei_p, ei_s, u3_out, aux_out,
             src_v, dst_v, w_v, inv_v, acc_v, tmp_v, idx_v, aux_v,
             sh_cnt, sh_u1, sh_u2, sh_u3):
    c = lax.axis_index("c")
    s = lax.axis_index("s")
    base = s * EC

    @pl.when(c == 0)
    def _():
        pltpu.sync_copy(ei_p.at[pl.ds(base, EC)], src_v)
        pltpu.sync_copy(ei_p.at[pl.ds(E + base, EC)], dst_v)

    @pl.when(c == 1)
    def _():
        pltpu.sync_copy(ei_s.at[pl.ds(base, EC)], src_v)
        pltpu.sync_copy(ei_s.at[pl.ds(E + base, EC)], dst_v)

    iota = lax.iota(jnp.int32, 16)
    for i in range(NROW // 16):
        idx_v[pl.ds(i * 16, 16)] = iota + i * 16

    _zero(acc_v)

    for tid, sh in ((0, sh_cnt), (1, sh_u1), (2, sh_u2), (3, sh_u3)):
        @pl.when(s == tid)
        def _():
            pltpu.sync_copy(acc_v, sh)
    plsc.subcore_barrier()

    ones = jnp.ones((16,), jnp.float32)

    @plsc.parallel_loop(0, VEC, unroll=UNROLL)
    def _(i):
        dv = dst_v[pl.ds(i * 16, 16)]
        plsc.addupdate_scatter(acc_v, [lax.shift_right_logical(dv, 7), dv & 127], ones)
    pltpu.sync_copy(acc_v, sh_cnt.at[idx_v], add=True)
    plsc.subcore_barrier()

    pltpu.sync_copy(sh_cnt, tmp_v)

    @plsc.parallel_loop(0, NROW)
    def _(r):
        for j in range(8):
            sl = pl.ds(j * 16, 16)
            inv_v[r, sl] = 1.0 / jnp.maximum(tmp_v[r, sl], 1.0)

    def edge_pass(w_ref, sh_dst):
        _zero(acc_v)

        @plsc.parallel_loop(0, VEC, unroll=UNROLL)
        def _(i):
            sl = pl.ds(i * 16, 16)
            dv = dst_v[sl]
            sv = src_v[sl]
            vals = plsc.load_gather(w_ref, [lax.shift_right_logical(dv, 7), dv & 127])
            plsc.addupdate_scatter(acc_v, [lax.shift_right_logical(sv, 7), sv & 127], vals)
        pltpu.sync_copy(acc_v, sh_dst.at[idx_v], add=True)
        plsc.subcore_barrier()

    def finish_pass(sh_src, last):
        pltpu.sync_copy(sh_src, tmp_v)

        @plsc.parallel_loop(0, NROW, carry=jnp.zeros((16,), jnp.float32))
        def acc(r, a):
            for j in range(8):
                sl = pl.ds(j * 16, 16)
                uv = tmp_v[r, sl]
                if not last:
                    w_v[r, sl] = uv * inv_v[r, sl]
                a = a + uv
            return a
        return lax.reduce_sum_p.bind(acc, axes=(0,))

    edge_pass(inv_v, sh_u1)
    s1 = finish_pass(sh_u1, last=False)
    edge_pass(w_v, sh_u2)
    s2 = finish_pass(sh_u2, last=False)
    edge_pass(w_v, sh_u3)

    @pl.when(s < NROW // 8)
    def _():
        pltpu.sync_copy(sh_u3.at[pl.ds(s * 8, 8)],
                        u3_out.at[c, pl.ds(s * 8, 8)])

    @pl.when(s == 0)
    def _():
        lane = lax.iota(jnp.int32, 16)
        vec = jnp.where(lane == 0, jnp.full((16,), s1, jnp.float32),
                        jnp.where(lane == 1, jnp.full((16,), s2, jnp.float32),
                                  jnp.zeros((16,), jnp.float32)))
        aux_v[...] = vec
        pltpu.sync_copy(aux_v, aux_out.at[pl.ds(c * 16, 16)])


def _sc_call(ei_p, ei_s):
    mesh = plsc.VectorSubcoreMesh(core_axis_name="c", subcore_axis_name="s")
    kern = functools.partial(
        pl.kernel,
        mesh=mesh,
        compiler_params=pltpu.CompilerParams(needs_layout_passes=False),
        out_type=(jax.ShapeDtypeStruct((2, NROW, 128), jnp.float32),
                  jax.ShapeDtypeStruct((32,), jnp.float32)),
        scratch_types=[
            pltpu.VMEM((EC,), jnp.int32),
            pltpu.VMEM((EC,), jnp.int32),
            pltpu.VMEM((NROW, 128), jnp.float32),
            pltpu.VMEM((NROW, 128), jnp.float32),
            pltpu.VMEM((NROW, 128), jnp.float32),
            pltpu.VMEM((NROW, 128), jnp.float32),
            pltpu.VMEM((NROW,), jnp.int32),
            pltpu.VMEM((16,), jnp.float32),
            pltpu.VMEM_SHARED((NROW, 128), jnp.float32),
            pltpu.VMEM_SHARED((NROW, 128), jnp.float32),
            pltpu.VMEM_SHARED((NROW, 128), jnp.float32),
            pltpu.VMEM_SHARED((NROW, 128), jnp.float32),
        ],
    )(_sc_body)
    return kern(ei_p, ei_s)


NBLK = 10
BLK = NPAD // NBLK


def _tc_body(u3_ref, fp_ref, fs_ref, aux_ref,
             W1_ref, b1_ref, W2_ref, b2_ref, W3_ref, b3_ref,
             Wr_ref, br_ref, Wm1_ref, bm1_ref, Wm2_ref, bm2_ref,
             out_ref, y_acc):
    i = pl.program_id(0)

    @pl.when(i == 0)
    def _():
        y_acc[...] = jnp.zeros_like(y_acc)

    u3 = u3_ref[...]
    yp = jnp.dot(u3[0:1, :], fp_ref[...], preferred_element_type=jnp.float32)
    ys = jnp.dot(u3[1:2, :], fs_ref[...], preferred_element_type=jnp.float32)
    y_acc[...] += jnp.concatenate([yp, ys], axis=0)

    @pl.when(i == NBLK - 1)
    def _():
        y = y_acc[...]
        aux = aux_ref[...]
        s1 = aux[:, 0:1]
        s2 = aux[:, 1:2]
        b1 = b1_ref[...]
        b2 = b2_ref[...]
        b3 = b3_ref[...]
        t = jnp.dot(y, W1_ref[...].T, preferred_element_type=jnp.float32) + s2 * b1
        t = jnp.dot(t, W2_ref[...].T, preferred_element_type=jnp.float32) + s1 * b2
        t = jnp.dot(t, W3_ref[...].T, preferred_element_type=jnp.float32)
        g = t * (1.0 / N) + b3
        z = jnp.dot(g, Wr_ref[...].T, preferred_element_type=jnp.float32) + br_ref[...]
        f = 1.0 / (1.0 + jnp.exp(-z))
        cat = jnp.concatenate([f[0:1, :], f[1:2, :]], axis=1)
        d1 = jnp.dot(cat, Wm1_ref[...].T, preferred_element_type=jnp.float32) + bm1_ref[...]
        d2 = jnp.sum(d1 * Wm2_ref[...], axis=1, keepdims=True) + bm2_ref[...]
        out_ref[...] = 1.0 / (1.0 + jnp.exp(-d2))


def _tc_call(u3r, fp_pad, fs_pad, aux, W1, b1, W2, b2, W3, b3, Wr, br, Wm1, bm1, Wm2, bm2):
    full = lambda shape: pl.BlockSpec(shape, lambda i: (0,) * len(shape))
    return pl.pallas_call(
        _tc_body,
        grid=(NBLK,),
        in_specs=[
            pl.BlockSpec((2, BLK), lambda i: (0, i)),
            pl.BlockSpec((BLK, D), lambda i: (i, 0)),
            pl.BlockSpec((BLK, D), lambda i: (i, 0)),
            full((2, 16)),
            full((D, D)), full((1, D)),
            full((D, D)), full((1, D)),
            full((D, D)), full((1, D)),
            full((D, D)), full((1, D)),
            full((D, 2 * D)), full((1, D)),
            full((1, D)), full((1, 1)),
        ],
        out_specs=pl.BlockSpec((1, 1), lambda i: (0, 0)),
        out_shape=jax.ShapeDtypeStruct((1, 1), jnp.float32),
        scratch_shapes=[pltpu.VMEM((2, D), jnp.float32)],
    )(u3r, fp_pad, fs_pad, aux, W1, b1, W2, b2, W3, b3, Wr, br, Wm1, bm1, Wm2, bm2)


def kernel(feat_p, feat_s, edge_index_p, edge_index_s,
           W1, b1, W2, b2, W3, b3, Wr, br, Wm1, bm1, Wm2, bm2):
    ei_p = edge_index_p.reshape(2 * E)
    ei_s = edge_index_s.reshape(2 * E)
    u3, aux = _sc_call(ei_p, ei_s)
    aux = aux.reshape(2, 16)
    u3r = u3.reshape(2, NPAD)
    pad = ((0, NPAD - N), (0, 0))
    fp_pad = jnp.pad(feat_p, pad)
    fs_pad = jnp.pad(feat_s, pad)
    out = _tc_call(u3r, fp_pad, fs_pad, aux,
                   W1, b1.reshape(1, D), W2, b2.reshape(1, D),
                   W3, b3.reshape(1, D), Wr, br.reshape(1, D),
                   Wm1, bm1.reshape(1, D), Wm2, bm2.reshape(1, 1))
    return out.reshape(1)


# speedup vs baseline: 104.8214x; 1.0046x over previous
import functools

import jax
import jax.numpy as jnp
from jax import lax
from jax.experimental import pallas as pl
from jax.experimental.pallas import tpu as pltpu
from jax.experimental.pallas import tpu_sc as plsc

N = 10000
E = 320000
D = 128
NROW = 80
NPAD = NROW * 128
NTILES = 16
EC = E // NTILES
VEC = EC // 16
CHUNK = NROW // NTILES
UNROLL = 5


def _zero(ref):
    @plsc.parallel_loop(0, NROW)
    def _(r):
        for j in range(8):
            ref[r, pl.ds(j * 16, 16)] = jnp.zeros((16,), jnp.float32)


def _sc_body(ei_p, ei_s, feat_p, feat_s, out_hbm,
             src_v, dst_v, w_v, inv_v, acc_v, tmp_v, idx_v,
             u3c_v, ybuf_v, yout_v, s1row_v, s2row_v,
             sh_cnt, sh_u1, sh_u2, sh_u3, sh_y):
    c = lax.axis_index("c")
    s = lax.axis_index("s")
    base = s * EC

    @pl.when(c == 0)
    def _():
        pltpu.sync_copy(ei_p.at[pl.ds(base, EC)], src_v)
        pltpu.sync_copy(ei_p.at[pl.ds(E + base, EC)], dst_v)

    @pl.when(c == 1)
    def _():
        pltpu.sync_copy(ei_s.at[pl.ds(base, EC)], src_v)
        pltpu.sync_copy(ei_s.at[pl.ds(E + base, EC)], dst_v)

    iota = lax.iota(jnp.int32, 16)
    for i in range(NROW // 16):
        idx_v[pl.ds(i * 16, 16)] = iota + i * 16

    _zero(acc_v)

    for tid, sh in ((0, sh_cnt), (1, sh_u1), (2, sh_u2), (3, sh_u3)):
        @pl.when(s == tid)
        def _():
            pltpu.sync_copy(acc_v, sh)
    plsc.subcore_barrier()

    ones = jnp.ones((16,), jnp.float32)

    @plsc.parallel_loop(0, VEC, unroll=UNROLL)
    def _(i):
        dv = dst_v[pl.ds(i * 16, 16)]
        plsc.addupdate_scatter(acc_v, [lax.shift_right_logical(dv, 7), dv & 127], ones)
    pltpu.sync_copy(acc_v, sh_cnt.at[idx_v], add=True)
    plsc.subcore_barrier()

    pltpu.sync_copy(sh_cnt, tmp_v)

    @plsc.parallel_loop(0, NROW)
    def _(r):
        for j in range(8):
            sl = pl.ds(j * 16, 16)
            inv_v[r, sl] = 1.0 / jnp.maximum(tmp_v[r, sl], 1.0)

    def edge_pass(w_ref, sh_dst):
        _zero(acc_v)

        @plsc.parallel_loop(0, VEC, unroll=UNROLL)
        def _(i):
            sl = pl.ds(i * 16, 16)
            dv = dst_v[sl]
            sv = src_v[sl]
            vals = plsc.load_gather(w_ref, [lax.shift_right_logical(dv, 7), dv & 127])
            plsc.addupdate_scatter(acc_v, [lax.shift_right_logical(sv, 7), sv & 127], vals)
        pltpu.sync_copy(acc_v, sh_dst.at[idx_v], add=True)
        plsc.subcore_barrier()

    def finish_pass(sh_src, last):
        pltpu.sync_copy(sh_src, tmp_v)

        @plsc.parallel_loop(0, NROW, carry=jnp.zeros((16,), jnp.float32))
        def acc(r, a):
            for j in range(8):
                sl = pl.ds(j * 16, 16)
                uv = tmp_v[r, sl]
                if not last:
                    w_v[r, sl] = uv * inv_v[r, sl]
                a = a + uv
            return a
        return lax.reduce_sum_p.bind(acc, axes=(0,))

    edge_pass(inv_v, sh_u1)
    s1 = finish_pass(sh_u1, last=False)
    edge_pass(w_v, sh_u2)
    s2 = finish_pass(sh_u2, last=False)
    edge_pass(w_v, sh_u3)

    pltpu.sync_copy(sh_u3.at[pl.ds(s * CHUNK, CHUNK)], u3c_v)
    for j in range(8):
        ybuf_v[0, pl.ds(j * 16, 16)] = jnp.zeros((16,), jnp.float32)

    def y_stage(feat_ref):
        for k in range(8):
            row0 = s * 640 + k * 80

            @pl.when(row0 < N)
            def _():
                pltpu.sync_copy(feat_ref.at[pl.ds(row0, 80), :], acc_v)
                zero8 = tuple(jnp.zeros((16,), jnp.float32) for _ in range(8))

                @plsc.parallel_loop(0, 80, carry=zero8)
                def yacc(r, yc):
                    n = k * 80 + r
                    u = plsc.load_gather(
                        u3c_v, [jnp.full((16,), lax.shift_right_logical(n, 7)),
                                jnp.full((16,), n & 127)])
                    return tuple(yc[i] + u * acc_v[r, pl.ds(i * 16, 16)]
                                 for i in range(8))
                for j in range(8):
                    plsc.addupdate(ybuf_v.at[0, pl.ds(j * 16, 16)], yacc[j])

    @pl.when(c == 0)
    def _():
        y_stage(feat_p)

    @pl.when(c == 1)
    def _():
        y_stage(feat_s)

    pltpu.sync_copy(ybuf_v, sh_y.at[pl.ds(s, 1), :])
    plsc.subcore_barrier()

    @pl.when(s == 0)
    def _():
        pltpu.sync_copy(sh_y, acc_v.at[pl.ds(0, 16), :])
        for j in range(8):
            tot = acc_v[0, pl.ds(j * 16, 16)]
            for i in range(1, 16):
                tot = tot + acc_v[i, pl.ds(j * 16, 16)]
            yout_v[pl.ds(j * 16, 16)] = tot
            s1row_v[pl.ds(j * 16, 16)] = jnp.full((16,), s1)
            s2row_v[pl.ds(j * 16, 16)] = jnp.full((16,), s2)
        pltpu.sync_copy(yout_v, out_hbm.at[pl.ds(c * 128, 128)])
        pltpu.sync_copy(s1row_v, out_hbm.at[pl.ds(256 + c * 128, 128)])
        pltpu.sync_copy(s2row_v, out_hbm.at[pl.ds(512 + c * 128, 128)])


def _sc_call(ei_p, ei_s, feat_p, feat_s):
    mesh = plsc.VectorSubcoreMesh(core_axis_name="c", subcore_axis_name="s")
    kern = functools.partial(
        pl.kernel,
        mesh=mesh,
        compiler_params=pltpu.CompilerParams(needs_layout_passes=False),
        out_type=jax.ShapeDtypeStruct((1024,), jnp.float32),
        scratch_types=[
            pltpu.VMEM((EC,), jnp.int32),
            pltpu.VMEM((EC,), jnp.int32),
            pltpu.VMEM((NROW, 128), jnp.float32),
            pltpu.VMEM((NROW, 128), jnp.float32),
            pltpu.VMEM((NROW, 128), jnp.float32),
            pltpu.VMEM((NROW, 128), jnp.float32),
            pltpu.VMEM((NROW,), jnp.int32),
            pltpu.VMEM((CHUNK, 128), jnp.float32),
            pltpu.VMEM((1, 128), jnp.float32),
            pltpu.VMEM((128,), jnp.float32),
            pltpu.VMEM((128,), jnp.float32),
            pltpu.VMEM((128,), jnp.float32),
            pltpu.VMEM_SHARED((NROW, 128), jnp.float32),
            pltpu.VMEM_SHARED((NROW, 128), jnp.float32),
            pltpu.VMEM_SHARED((NROW, 128), jnp.float32),
            pltpu.VMEM_SHARED((NROW, 128), jnp.float32),
            pltpu.VMEM_SHARED((16, 128), jnp.float32),
        ],
    )(_sc_body)
    return kern(ei_p, ei_s, feat_p, feat_s)


def _tc_body(o_ref,
             W1_ref, b1_ref, W2_ref, b2_ref, W3_ref, b3_ref,
             Wr_ref, br_ref, Wm1_ref, bm1_ref, Wm2_ref, bm2_ref,
             out_ref):
    two = lambda off: jnp.concatenate(
        [o_ref[pl.ds(off, 128)].reshape(1, D),
         o_ref[pl.ds(off + 128, 128)].reshape(1, D)], axis=0)
    y = two(0)
    S1 = two(256)
    S2 = two(512)
    t = jnp.dot(y, W1_ref[...].T, preferred_element_type=jnp.float32) + S2 * b1_ref[...]
    t = jnp.dot(t, W2_ref[...].T, preferred_element_type=jnp.float32) + S1 * b2_ref[...]
    t = jnp.dot(t, W3_ref[...].T, preferred_element_type=jnp.float32)
    g = t * (1.0 / N) + b3_ref[...]
    z = jnp.dot(g, Wr_ref[...].T, preferred_element_type=jnp.float32) + br_ref[...]
    f = 1.0 / (1.0 + jnp.exp(-z))
    cat = jnp.concatenate([f[0:1, :], f[1:2, :]], axis=1)
    d1 = jnp.dot(cat, Wm1_ref[...].T, preferred_element_type=jnp.float32) + bm1_ref[...]
    d2 = jnp.sum(d1 * Wm2_ref[...], axis=1, keepdims=True) + bm2_ref[...]
    out_ref[...] = 1.0 / (1.0 + jnp.exp(-d2))


def _tc_call(sc_out, W1, b1, W2, b2, W3, b3, Wr, br, Wm1, bm1, Wm2, bm2):
    return pl.pallas_call(
        _tc_body,
        out_shape=jax.ShapeDtypeStruct((1, 1), jnp.float32),
    )(sc_out, W1, b1, W2, b2, W3, b3, Wr, br, Wm1, bm1, Wm2, bm2)


def kernel(feat_p, feat_s, edge_index_p, edge_index_s,
           W1, b1, W2, b2, W3, b3, Wr, br, Wm1, bm1, Wm2, bm2):
    ei_p = edge_index_p.reshape(2 * E)
    ei_s = edge_index_s.reshape(2 * E)
    sc_out = _sc_call(ei_p, ei_s, feat_p, feat_s)
    out = _tc_call(sc_out,
                   W1, b1.reshape(1, D), W2, b2.reshape(1, D),
                   W3, b3.reshape(1, D), Wr, br.reshape(1, D),
                   Wm1, bm1.reshape(1, D), Wm2, bm2.reshape(1, 1))
    return out.reshape(1)


# speedup vs baseline: 117.1131x; 1.1173x over previous
import functools

import jax
import jax.numpy as jnp
from jax import lax
from jax.experimental import pallas as pl
from jax.experimental.pallas import tpu as pltpu
from jax.experimental.pallas import tpu_sc as plsc

N = 10000
E = 320000
D = 128
NROW = 80
NPAD = NROW * 128
NTILES = 16
GRP = (E // 128) // NTILES
CHUNK = NROW // NTILES


def _zero(ref):
    @plsc.parallel_loop(0, NROW)
    def _(r):
        for j in range(8):
            ref[r, pl.ds(j * 16, 16)] = jnp.zeros((16,), jnp.float32)


def _sc_body(ei_p, ei_s, feat_p, feat_s, out_hbm,
             ei_v, w_v, inv_v, acc_v, tmp_v, idx_v,
             u3c_v, ybuf_v, yout_v, s1row_v, s2row_v,
             sh_cnt, sh_u1, sh_u2, sh_u3, sh_y, ysem0, ysem1):
    c = lax.axis_index("c")
    s = lax.axis_index("s")

    ng = jnp.where(s < 4, GRP + 1, GRP)
    g0 = s * GRP + jnp.minimum(s, 4)

    def stage(ref):
        col0 = pl.multiple_of(g0 * 128, 128)

        @pl.when(s < 4)
        def _():
            pltpu.sync_copy(ref.at[:, pl.ds(col0, (GRP + 1) * 128)], ei_v)

        @pl.when(s >= 4)
        def _():
            pltpu.sync_copy(ref.at[:, pl.ds(col0, GRP * 128)],
                            ei_v.at[:, pl.ds(0, GRP * 128)])

    @pl.when(c == 0)
    def _():
        stage(ei_p)

    @pl.when(c == 1)
    def _():
        stage(ei_s)

    iota = lax.iota(jnp.int32, 16)
    for i in range(NROW // 16):
        idx_v[pl.ds(i * 16, 16)] = iota + i * 16

    _zero(acc_v)

    for tid, sh in ((0, sh_cnt), (1, sh_u1), (2, sh_u2), (3, sh_u3)):
        @pl.when(s == tid)
        def _():
            pltpu.sync_copy(acc_v, sh)
    plsc.subcore_barrier()

    ones = jnp.ones((16,), jnp.float32)

    @plsc.parallel_loop(0, ng)
    def _(g):
        for u in range(8):
            dv = ei_v[1, pl.ds(g * 128 + u * 16, 16)]
            plsc.addupdate_scatter(
                acc_v, [lax.shift_right_logical(dv, 7), dv & 127], ones)
    pltpu.sync_copy(acc_v, sh_cnt.at[idx_v], add=True)
    plsc.subcore_barrier()

    pltpu.sync_copy(sh_cnt, tmp_v)

    @plsc.parallel_loop(0, NROW)
    def _(r):
        for j in range(8):
            sl = pl.ds(j * 16, 16)
            inv_v[r, sl] = 1.0 / jnp.maximum(tmp_v[r, sl], 1.0)

    def edge_pass(w_ref, sh_dst):
        _zero(acc_v)

        @plsc.parallel_loop(0, ng)
        def _(g):
            for u in range(8):
                sl = pl.ds(g * 128 + u * 16, 16)
                dv = ei_v[1, sl]
                sv = ei_v[0, sl]
                vals = plsc.load_gather(
                    w_ref, [lax.shift_right_logical(dv, 7), dv & 127])
                plsc.addupdate_scatter(
                    acc_v, [lax.shift_right_logical(sv, 7), sv & 127], vals)
        pltpu.sync_copy(acc_v, sh_dst.at[idx_v], add=True)
        plsc.subcore_barrier()

    def finish_pass(sh_src, last):
        pltpu.sync_copy(sh_src, tmp_v)

        @plsc.parallel_loop(0, NROW, carry=jnp.zeros((16,), jnp.float32))
        def acc(r, a):
            for j in range(8):
                sl = pl.ds(j * 16, 16)
                uv = tmp_v[r, sl]
                if not last:
                    w_v[r, sl] = uv * inv_v[r, sl]
                a = a + uv
            return a
        return lax.reduce_sum_p.bind(acc, axes=(0,))

    edge_pass(inv_v, sh_u1)
    s1 = finish_pass(sh_u1, last=False)
    edge_pass(w_v, sh_u2)
    s2 = finish_pass(sh_u2, last=False)
    edge_pass(w_v, sh_u3)

    pltpu.sync_copy(sh_u3.at[pl.ds(s * CHUNK, CHUNK)], u3c_v)
    for j in range(8):
        ybuf_v[0, pl.ds(j * 16, 16)] = jnp.zeros((16,), jnp.float32)

    def y_stage(feat_ref, sem0, sem1):
        bufs = (acc_v, tmp_v)
        sems = (sem0, sem1)

        def copy(k):
            return pltpu.make_async_copy(
                feat_ref.at[pl.ds(s * 640 + k * 80, 80), :],
                bufs[k % 2], sems[k % 2])

        copy(0).start()
        for k in range(8):
            row0 = s * 640 + k * 80

            @pl.when(row0 < N)
            def _():
                copy(k).wait()
                if k < 7:
                    @pl.when(row0 + 80 < N)
                    def _():
                        copy(k + 1).start()
                buf = bufs[k % 2]
                zero8 = tuple(jnp.zeros((16,), jnp.float32) for _ in range(8))

                @plsc.parallel_loop(0, 80, carry=zero8)
                def yacc(r, yc):
                    n = k * 80 + r
                    u = plsc.load_gather(
                        u3c_v, [jnp.full((16,), lax.shift_right_logical(n, 7)),
                                jnp.full((16,), n & 127)])
                    return tuple(yc[i] + u * buf[r, pl.ds(i * 16, 16)]
                                 for i in range(8))
                for j in range(8):
                    plsc.addupdate(ybuf_v.at[0, pl.ds(j * 16, 16)], yacc[j])

    @pl.when(c == 0)
    def _():
        y_stage(feat_p, ysem0, ysem1)

    @pl.when(c == 1)
    def _():
        y_stage(feat_s, ysem0, ysem1)

    pltpu.sync_copy(ybuf_v, sh_y.at[pl.ds(s, 1), :])
    plsc.subcore_barrier()

    @pl.when(s == 0)
    def _():
        pltpu.sync_copy(sh_y, acc_v.at[pl.ds(0, 16), :])
        for j in range(8):
            tot = acc_v[0, pl.ds(j * 16, 16)]
            for i in range(1, 16):
                tot = tot + acc_v[i, pl.ds(j * 16, 16)]
            yout_v[pl.ds(j * 16, 16)] = tot
            s1row_v[pl.ds(j * 16, 16)] = jnp.full((16,), s1)
            s2row_v[pl.ds(j * 16, 16)] = jnp.full((16,), s2)
        pltpu.sync_copy(yout_v, out_hbm.at[pl.ds(c * 128, 128)])
        pltpu.sync_copy(s1row_v, out_hbm.at[pl.ds(256 + c * 128, 128)])
        pltpu.sync_copy(s2row_v, out_hbm.at[pl.ds(512 + c * 128, 128)])


def _sc_call(ei_p, ei_s, feat_p, feat_s):
    mesh = plsc.VectorSubcoreMesh(core_axis_name="c", subcore_axis_name="s")
    kern = functools.partial(
        pl.kernel,
        mesh=mesh,
        compiler_params=pltpu.CompilerParams(needs_layout_passes=False),
        out_type=jax.ShapeDtypeStruct((1024,), jnp.float32),
        scratch_types=[
            pltpu.VMEM((2, (GRP + 1) * 128), jnp.int32),
            pltpu.VMEM((NROW, 128), jnp.float32),
            pltpu.VMEM((NROW, 128), jnp.float32),
            pltpu.VMEM((NROW, 128), jnp.float32),
            pltpu.VMEM((NROW, 128), jnp.float32),
            pltpu.VMEM((NROW,), jnp.int32),
            pltpu.VMEM((CHUNK, 128), jnp.float32),
            pltpu.VMEM((1, 128), jnp.float32),
            pltpu.VMEM((128,), jnp.float32),
            pltpu.VMEM((128,), jnp.float32),
            pltpu.VMEM((128,), jnp.float32),
            pltpu.VMEM_SHARED((NROW, 128), jnp.float32),
            pltpu.VMEM_SHARED((NROW, 128), jnp.float32),
            pltpu.VMEM_SHARED((NROW, 128), jnp.float32),
            pltpu.VMEM_SHARED((NROW, 128), jnp.float32),
            pltpu.VMEM_SHARED((16, 128), jnp.float32),
            pltpu.SemaphoreType.DMA,
            pltpu.SemaphoreType.DMA,
        ],
    )(_sc_body)
    return kern(ei_p, ei_s, feat_p, feat_s)


def _tc_body(o_ref,
             W1_ref, b1_ref, W2_ref, b2_ref, W3_ref, b3_ref,
             Wr_ref, br_ref, Wm1_ref, bm1_ref, Wm2_ref, bm2_ref,
             out_ref):
    two = lambda off: jnp.concatenate(
        [o_ref[pl.ds(off, 128)].reshape(1, D),
         o_ref[pl.ds(off + 128, 128)].reshape(1, D)], axis=0)
    y = two(0)
    S1 = two(256)
    S2 = two(512)
    t = jnp.dot(y, W1_ref[...].T, preferred_element_type=jnp.float32) + S2 * b1_ref[...]
    t = jnp.dot(t, W2_ref[...].T, preferred_element_type=jnp.float32) + S1 * b2_ref[...]
    t = jnp.dot(t, W3_ref[...].T, preferred_element_type=jnp.float32)
    g = t * (1.0 / N) + b3_ref[...]
    z = jnp.dot(g, Wr_ref[...].T, preferred_element_type=jnp.float32) + br_ref[...]
    f = 1.0 / (1.0 + jnp.exp(-z))
    cat = jnp.concatenate([f[0:1, :], f[1:2, :]], axis=1)
    d1 = jnp.dot(cat, Wm1_ref[...].T, preferred_element_type=jnp.float32) + bm1_ref[...]
    d2 = jnp.sum(d1 * Wm2_ref[...], axis=1, keepdims=True) + bm2_ref[...]
    out_ref[...] = 1.0 / (1.0 + jnp.exp(-d2))


def _tc_call(sc_out, W1, b1, W2, b2, W3, b3, Wr, br, Wm1, bm1, Wm2, bm2):
    return pl.pallas_call(
        _tc_body,
        out_shape=jax.ShapeDtypeStruct((1, 1), jnp.float32),
    )(sc_out, W1, b1, W2, b2, W3, b3, Wr, br, Wm1, bm1, Wm2, bm2)


def kernel(feat_p, feat_s, edge_index_p, edge_index_s,
           W1, b1, W2, b2, W3, b3, Wr, br, Wm1, bm1, Wm2, bm2):
    sc_out = _sc_call(edge_index_p, edge_index_s, feat_p, feat_s)
    out = _tc_call(sc_out,
                   W1, b1.reshape(1, D), W2, b2.reshape(1, D),
                   W3, b3.reshape(1, D), Wr, br.reshape(1, D),
                   Wm1, bm1.reshape(1, D), Wm2, bm2.reshape(1, 1))
    return out.reshape(1)


# speedup vs baseline: 117.3282x; 1.0018x over previous
import functools

import jax
import jax.numpy as jnp
from jax import lax
from jax.experimental import pallas as pl
from jax.experimental.pallas import tpu as pltpu
from jax.experimental.pallas import tpu_sc as plsc

N = 10000
E = 320000
D = 128
NROW = 80
NPAD = NROW * 128
NTILES = 16
GRP = (E // 128) // NTILES
CHUNK = NROW // NTILES


def _zero(ref):
    @plsc.parallel_loop(0, NROW)
    def _(r):
        for j in range(8):
            ref[r, pl.ds(j * 16, 16)] = jnp.zeros((16,), jnp.float32)


def _sc_body(ei_p, ei_s, feat_p, feat_s, out_hbm,
             ei_v, w_v, inv_v, acc_v, tmp_v, idx_v,
             u3c_v, ybuf_v, yout_v, s1row_v, s2row_v,
             sh_cnt, sh_u1, sh_u2, sh_u3, sh_y, ysem0, ysem1):
    c = lax.axis_index("c")
    s = lax.axis_index("s")

    ng = jnp.where(s < 4, GRP + 1, GRP)
    g0 = s * GRP + jnp.minimum(s, 4)

    def stage(ref, act):
        col0 = pl.multiple_of(g0 * 128, 128)
        big = pltpu.make_async_copy(
            ref.at[:, pl.ds(col0, (GRP + 1) * 128)], ei_v, ysem0)
        small = pltpu.make_async_copy(
            ref.at[:, pl.ds(col0, GRP * 128)],
            ei_v.at[:, pl.ds(0, GRP * 128)], ysem0)

        @pl.when(s < 4)
        def _():
            getattr(big, act)()

        @pl.when(s >= 4)
        def _():
            getattr(small, act)()

    def per_core(fn):
        @pl.when(c == 0)
        def _():
            fn(ei_p)

        @pl.when(c == 1)
        def _():
            fn(ei_s)

    per_core(lambda ref: stage(ref, "start"))

    iota = lax.iota(jnp.int32, 16)
    for i in range(NROW // 16):
        idx_v[pl.ds(i * 16, 16)] = iota + i * 16

    _zero(acc_v)

    for tid, sh in ((0, sh_cnt), (1, sh_u1), (2, sh_u2), (3, sh_u3)):
        @pl.when(s == tid)
        def _():
            pltpu.sync_copy(acc_v, sh)

    per_core(lambda ref: stage(ref, "wait"))

    ones = jnp.ones((16,), jnp.float32)

    @plsc.parallel_loop(0, ng, unroll=2)
    def _(g):
        for u in range(8):
            dv = ei_v[1, pl.ds(g * 128 + u * 16, 16)]
            plsc.addupdate_scatter(
                acc_v, [lax.shift_right_logical(dv, 7), dv & 127], ones)
    plsc.subcore_barrier()
    pltpu.sync_copy(acc_v, sh_cnt.at[idx_v], add=True)
    plsc.subcore_barrier()

    pltpu.sync_copy(sh_cnt, tmp_v)

    @plsc.parallel_loop(0, NROW)
    def _(r):
        for j in range(8):
            sl = pl.ds(j * 16, 16)
            inv_v[r, sl] = 1.0 / jnp.maximum(tmp_v[r, sl], 1.0)

    def edge_pass(w_ref, sh_dst):
        _zero(acc_v)

        @plsc.parallel_loop(0, ng, unroll=2)
        def _(g):
            for u in range(8):
                sl = pl.ds(g * 128 + u * 16, 16)
                dv = ei_v[1, sl]
                sv = ei_v[0, sl]
                vals = plsc.load_gather(
                    w_ref, [lax.shift_right_logical(dv, 7), dv & 127])
                plsc.addupdate_scatter(
                    acc_v, [lax.shift_right_logical(sv, 7), sv & 127], vals)
        pltpu.sync_copy(acc_v, sh_dst.at[idx_v], add=True)
        plsc.subcore_barrier()

    def finish_pass(sh_src, last):
        pltpu.sync_copy(sh_src, tmp_v)

        @plsc.parallel_loop(0, NROW, carry=jnp.zeros((16,), jnp.float32))
        def acc(r, a):
            for j in range(8):
                sl = pl.ds(j * 16, 16)
                uv = tmp_v[r, sl]
                if not last:
                    w_v[r, sl] = uv * inv_v[r, sl]
                a = a + uv
            return a
        return lax.reduce_sum_p.bind(acc, axes=(0,))

    edge_pass(inv_v, sh_u1)
    s1 = finish_pass(sh_u1, last=False)
    edge_pass(w_v, sh_u2)
    s2 = finish_pass(sh_u2, last=False)
    edge_pass(w_v, sh_u3)

    pltpu.sync_copy(sh_u3.at[pl.ds(s * CHUNK, CHUNK)], u3c_v)
    for j in range(8):
        ybuf_v[0, pl.ds(j * 16, 16)] = jnp.zeros((16,), jnp.float32)

    def y_stage(feat_ref, sem0, sem1):
        bufs = (acc_v, tmp_v)
        sems = (sem0, sem1)

        def copy(k):
            return pltpu.make_async_copy(
                feat_ref.at[pl.ds(s * 640 + k * 80, 80), :],
                bufs[k % 2], sems[k % 2])

        copy(0).start()
        for k in range(8):
            row0 = s * 640 + k * 80

            @pl.when(row0 < N)
            def _():
                copy(k).wait()
                if k < 7:
                    @pl.when(row0 + 80 < N)
                    def _():
                        copy(k + 1).start()
                buf = bufs[k % 2]
                zero8 = tuple(jnp.zeros((16,), jnp.float32) for _ in range(8))

                @plsc.parallel_loop(0, 80, carry=zero8)
                def yacc(r, yc):
                    n = k * 80 + r
                    u = plsc.load_gather(
                        u3c_v, [jnp.full((16,), lax.shift_right_logical(n, 7)),
                                jnp.full((16,), n & 127)])
                    return tuple(yc[i] + u * buf[r, pl.ds(i * 16, 16)]
                                 for i in range(8))
                for j in range(8):
                    plsc.addupdate(ybuf_v.at[0, pl.ds(j * 16, 16)], yacc[j])

    @pl.when(c == 0)
    def _():
        y_stage(feat_p, ysem0, ysem1)

    @pl.when(c == 1)
    def _():
        y_stage(feat_s, ysem0, ysem1)

    pltpu.sync_copy(ybuf_v, sh_y.at[pl.ds(s, 1), :])
    plsc.subcore_barrier()

    @pl.when(s == 0)
    def _():
        pltpu.sync_copy(sh_y, acc_v.at[pl.ds(0, 16), :])
        for j in range(8):
            tot = acc_v[0, pl.ds(j * 16, 16)]
            for i in range(1, 16):
                tot = tot + acc_v[i, pl.ds(j * 16, 16)]
            yout_v[pl.ds(j * 16, 16)] = tot
            s1row_v[pl.ds(j * 16, 16)] = jnp.full((16,), s1)
            s2row_v[pl.ds(j * 16, 16)] = jnp.full((16,), s2)
        pltpu.sync_copy(yout_v, out_hbm.at[pl.ds(c * 128, 128)])
        pltpu.sync_copy(s1row_v, out_hbm.at[pl.ds(256 + c * 128, 128)])
        pltpu.sync_copy(s2row_v, out_hbm.at[pl.ds(512 + c * 128, 128)])


def _sc_call(ei_p, ei_s, feat_p, feat_s):
    mesh = plsc.VectorSubcoreMesh(core_axis_name="c", subcore_axis_name="s")
    kern = functools.partial(
        pl.kernel,
        mesh=mesh,
        compiler_params=pltpu.CompilerParams(needs_layout_passes=False),
        out_type=jax.ShapeDtypeStruct((1024,), jnp.float32),
        scratch_types=[
            pltpu.VMEM((2, (GRP + 1) * 128), jnp.int32),
            pltpu.VMEM((NROW, 128), jnp.float32),
            pltpu.VMEM((NROW, 128), jnp.float32),
            pltpu.VMEM((NROW, 128), jnp.float32),
            pltpu.VMEM((NROW, 128), jnp.float32),
            pltpu.VMEM((NROW,), jnp.int32),
            pltpu.VMEM((CHUNK, 128), jnp.float32),
            pltpu.VMEM((1, 128), jnp.float32),
            pltpu.VMEM((128,), jnp.float32),
            pltpu.VMEM((128,), jnp.float32),
            pltpu.VMEM((128,), jnp.float32),
            pltpu.VMEM_SHARED((NROW, 128), jnp.float32),
            pltpu.VMEM_SHARED((NROW, 128), jnp.float32),
            pltpu.VMEM_SHARED((NROW, 128), jnp.float32),
            pltpu.VMEM_SHARED((NROW, 128), jnp.float32),
            pltpu.VMEM_SHARED((16, 128), jnp.float32),
            pltpu.SemaphoreType.DMA,
            pltpu.SemaphoreType.DMA,
        ],
    )(_sc_body)
    return kern(ei_p, ei_s, feat_p, feat_s)


def _tc_body(o_ref,
             W1_ref, b1_ref, W2_ref, b2_ref, W3_ref, b3_ref,
             Wr_ref, br_ref, Wm1_ref, bm1_ref, Wm2_ref, bm2_ref,
             out_ref):
    two = lambda off: jnp.concatenate(
        [o_ref[pl.ds(off, 128)].reshape(1, D),
         o_ref[pl.ds(off + 128, 128)].reshape(1, D)], axis=0)
    y = two(0)
    S1 = two(256)
    S2 = two(512)
    t = jnp.dot(y, W1_ref[...].T, preferred_element_type=jnp.float32) + S2 * b1_ref[...]
    t = jnp.dot(t, W2_ref[...].T, preferred_element_type=jnp.float32) + S1 * b2_ref[...]
    t = jnp.dot(t, W3_ref[...].T, preferred_element_type=jnp.float32)
    g = t * (1.0 / N) + b3_ref[...]
    z = jnp.dot(g, Wr_ref[...].T, preferred_element_type=jnp.float32) + br_ref[...]
    f = 1.0 / (1.0 + jnp.exp(-z))
    cat = jnp.concatenate([f[0:1, :], f[1:2, :]], axis=1)
    d1 = jnp.dot(cat, Wm1_ref[...].T, preferred_element_type=jnp.float32) + bm1_ref[...]
    d2 = jnp.sum(d1 * Wm2_ref[...], axis=1, keepdims=True) + bm2_ref[...]
    out_ref[...] = 1.0 / (1.0 + jnp.exp(-d2))


def _tc_call(sc_out, W1, b1, W2, b2, W3, b3, Wr, br, Wm1, bm1, Wm2, bm2):
    return pl.pallas_call(
        _tc_body,
        out_shape=jax.ShapeDtypeStruct((1, 1), jnp.float32),
    )(sc_out, W1, b1, W2, b2, W3, b3, Wr, br, Wm1, bm1, Wm2, bm2)


def kernel(feat_p, feat_s, edge_index_p, edge_index_s,
           W1, b1, W2, b2, W3, b3, Wr, br, Wm1, bm1, Wm2, bm2):
    sc_out = _sc_call(edge_index_p, edge_index_s, feat_p, feat_s)
    out = _tc_call(sc_out,
                   W1, b1.reshape(1, D), W2, b2.reshape(1, D),
                   W3, b3.reshape(1, D), Wr, br.reshape(1, D),
                   Wm1, bm1.reshape(1, D), Wm2, bm2.reshape(1, 1))
    return out.reshape(1)


# speedup vs baseline: 118.2706x; 1.0080x over previous
import functools

import jax
import jax.numpy as jnp
from jax import lax
from jax.experimental import pallas as pl
from jax.experimental.pallas import tpu as pltpu
from jax.experimental.pallas import tpu_sc as plsc

N = 10000
E = 320000
D = 128
NROW = 80
NPAD = NROW * 128
NTILES = 16
GRP = (E // 128) // NTILES
CHUNK = NROW // NTILES


def _zero(ref):
    @plsc.parallel_loop(0, NROW)
    def _(r):
        for j in range(8):
            ref[r, pl.ds(j * 16, 16)] = jnp.zeros((16,), jnp.float32)


def _sc_body(ei_p, ei_s, feat_p, feat_s, out_hbm,
             ei_v, w_v, inv_v, acc_v, tmp_v, idx_v,
             u3c_v, ybuf_v, yout_v, s1row_v, s2row_v,
             sh_cnt, sh_u1, sh_u2, sh_u3, sh_y, ysem0, ysem1):
    c = lax.axis_index("c")
    s = lax.axis_index("s")

    ng = jnp.where(s < 4, GRP + 1, GRP)
    g0 = s * GRP + jnp.minimum(s, 4)

    def stage(ref, act):
        col0 = pl.multiple_of(g0 * 128, 128)
        big = pltpu.make_async_copy(
            ref.at[:, pl.ds(col0, (GRP + 1) * 128)], ei_v, ysem0)
        small = pltpu.make_async_copy(
            ref.at[:, pl.ds(col0, GRP * 128)],
            ei_v.at[:, pl.ds(0, GRP * 128)], ysem0)

        @pl.when(s < 4)
        def _():
            getattr(big, act)()

        @pl.when(s >= 4)
        def _():
            getattr(small, act)()

    def per_core(fn):
        @pl.when(c == 0)
        def _():
            fn(ei_p)

        @pl.when(c == 1)
        def _():
            fn(ei_s)

    per_core(lambda ref: stage(ref, "start"))

    iota = lax.iota(jnp.int32, 16)
    for i in range(NROW // 16):
        idx_v[pl.ds(i * 16, 16)] = iota + i * 16

    _zero(acc_v)

    for tid, sh in ((0, sh_cnt), (1, sh_u1), (2, sh_u2), (3, sh_u3)):
        @pl.when(s == tid)
        def _():
            pltpu.sync_copy(acc_v, sh)

    per_core(lambda ref: stage(ref, "wait"))

    ones = jnp.ones((16,), jnp.float32)

    @plsc.parallel_loop(0, ng, unroll=2)
    def _(g):
        for u in range(8):
            dv = ei_v[1, pl.ds(g * 128 + u * 16, 16)]
            plsc.addupdate_scatter(
                acc_v, [lax.shift_right_logical(dv, 7), dv & 127], ones)
    plsc.subcore_barrier()
    pltpu.sync_copy(acc_v, sh_cnt.at[idx_v], add=True)
    plsc.subcore_barrier()

    pltpu.sync_copy(sh_cnt, tmp_v)

    @plsc.parallel_loop(0, NROW)
    def _(r):
        for j in range(8):
            sl = pl.ds(j * 16, 16)
            inv_v[r, sl] = 1.0 / jnp.maximum(tmp_v[r, sl], 1.0)

    def edge_pass(w_ref, sh_dst):
        _zero(acc_v)

        @plsc.parallel_loop(0, ng, unroll=2)
        def _(g):
            for u in range(8):
                sl = pl.ds(g * 128 + u * 16, 16)
                dv = ei_v[1, sl]
                sv = ei_v[0, sl]
                vals = plsc.load_gather(
                    w_ref, [lax.shift_right_logical(dv, 7), dv & 127])
                plsc.addupdate_scatter(
                    acc_v, [lax.shift_right_logical(sv, 7), sv & 127], vals)
        pltpu.sync_copy(acc_v, sh_dst.at[idx_v], add=True)
        plsc.subcore_barrier()

    def finish_pass(sh_src, last):
        pltpu.sync_copy(sh_src, tmp_v)

        @plsc.parallel_loop(0, NROW, carry=jnp.zeros((16,), jnp.float32))
        def acc(r, a):
            for j in range(8):
                sl = pl.ds(j * 16, 16)
                uv = tmp_v[r, sl]
                if not last:
                    w_v[r, sl] = uv * inv_v[r, sl]
                a = a + uv
            return a
        return lax.reduce_sum_p.bind(acc, axes=(0,))

    edge_pass(inv_v, sh_u1)
    s1 = finish_pass(sh_u1, last=False)
    edge_pass(w_v, sh_u2)
    s2 = finish_pass(sh_u2, last=False)
    edge_pass(w_v, sh_u3)

    pltpu.sync_copy(sh_u3.at[pl.ds(s * CHUNK, CHUNK)], u3c_v)
    for j in range(8):
        ybuf_v[0, pl.ds(j * 16, 16)] = jnp.zeros((16,), jnp.float32)

    def y_stage(feat_ref, sem0, sem1):
        bufs = (acc_v, tmp_v)
        sems = (sem0, sem1)

        def copy(k):
            return pltpu.make_async_copy(
                feat_ref.at[pl.ds(s * 640 + k * 80, 80), :],
                bufs[k % 2], sems[k % 2])

        copy(0).start()
        for k in range(8):
            row0 = s * 640 + k * 80

            @pl.when(row0 < N)
            def _():
                copy(k).wait()
                if k < 7:
                    @pl.when(row0 + 80 < N)
                    def _():
                        copy(k + 1).start()
                buf = bufs[k % 2]
                zero8 = tuple(jnp.zeros((16,), jnp.float32) for _ in range(8))

                @plsc.parallel_loop(0, 80, carry=zero8)
                def yacc(r, yc):
                    n = k * 80 + r
                    uvec = u3c_v[lax.shift_right_logical(n, 7),
                                 pl.ds(n & 0x70, 16)]
                    u = jnp.take_along_axis(
                        uvec, jnp.full((16,), n & 15, jnp.int32), axis=0)
                    return tuple(yc[i] + u * buf[r, pl.ds(i * 16, 16)]
                                 for i in range(8))
                for j in range(8):
                    plsc.addupdate(ybuf_v.at[0, pl.ds(j * 16, 16)], yacc[j])

    @pl.when(c == 0)
    def _():
        y_stage(feat_p, ysem0, ysem1)

    @pl.when(c == 1)
    def _():
        y_stage(feat_s, ysem0, ysem1)

    pltpu.sync_copy(ybuf_v, sh_y.at[pl.ds(s, 1), :])
    plsc.subcore_barrier()

    @pl.when(s == 0)
    def _():
        pltpu.sync_copy(sh_y, acc_v.at[pl.ds(0, 16), :])
        for j in range(8):
            tot = acc_v[0, pl.ds(j * 16, 16)]
            for i in range(1, 16):
                tot = tot + acc_v[i, pl.ds(j * 16, 16)]
            yout_v[pl.ds(j * 16, 16)] = tot
            s1row_v[pl.ds(j * 16, 16)] = jnp.full((16,), s1)
            s2row_v[pl.ds(j * 16, 16)] = jnp.full((16,), s2)
        pltpu.sync_copy(yout_v, out_hbm.at[pl.ds(c * 128, 128)])
        pltpu.sync_copy(s1row_v, out_hbm.at[pl.ds(256 + c * 128, 128)])
        pltpu.sync_copy(s2row_v, out_hbm.at[pl.ds(512 + c * 128, 128)])


def _sc_call(ei_p, ei_s, feat_p, feat_s):
    mesh = plsc.VectorSubcoreMesh(core_axis_name="c", subcore_axis_name="s")
    kern = functools.partial(
        pl.kernel,
        mesh=mesh,
        compiler_params=pltpu.CompilerParams(needs_layout_passes=False),
        out_type=jax.ShapeDtypeStruct((1024,), jnp.float32),
        scratch_types=[
            pltpu.VMEM((2, (GRP + 1) * 128), jnp.int32),
            pltpu.VMEM((NROW, 128), jnp.float32),
            pltpu.VMEM((NROW, 128), jnp.float32),
            pltpu.VMEM((NROW, 128), jnp.float32),
            pltpu.VMEM((NROW, 128), jnp.float32),
            pltpu.VMEM((NROW,), jnp.int32),
            pltpu.VMEM((CHUNK, 128), jnp.float32),
            pltpu.VMEM((1, 128), jnp.float32),
            pltpu.VMEM((128,), jnp.float32),
            pltpu.VMEM((128,), jnp.float32),
            pltpu.VMEM((128,), jnp.float32),
            pltpu.VMEM_SHARED((NROW, 128), jnp.float32),
            pltpu.VMEM_SHARED((NROW, 128), jnp.float32),
            pltpu.VMEM_SHARED((NROW, 128), jnp.float32),
            pltpu.VMEM_SHARED((NROW, 128), jnp.float32),
            pltpu.VMEM_SHARED((16, 128), jnp.float32),
            pltpu.SemaphoreType.DMA,
            pltpu.SemaphoreType.DMA,
        ],
    )(_sc_body)
    return kern(ei_p, ei_s, feat_p, feat_s)


def _tc_body(o_ref,
             W1_ref, b1_ref, W2_ref, b2_ref, W3_ref, b3_ref,
             Wr_ref, br_ref, Wm1_ref, bm1_ref, Wm2_ref, bm2_ref,
             out_ref):
    two = lambda off: jnp.concatenate(
        [o_ref[pl.ds(off, 128)].reshape(1, D),
         o_ref[pl.ds(off + 128, 128)].reshape(1, D)], axis=0)
    y = two(0)
    S1 = two(256)
    S2 = two(512)
    t = jnp.dot(y, W1_ref[...].T, preferred_element_type=jnp.float32) + S2 * b1_ref[...]
    t = jnp.dot(t, W2_ref[...].T, preferred_element_type=jnp.float32) + S1 * b2_ref[...]
    t = jnp.dot(t, W3_ref[...].T, preferred_element_type=jnp.float32)
    g = t * (1.0 / N) + b3_ref[...]
    z = jnp.dot(g, Wr_ref[...].T, preferred_element_type=jnp.float32) + br_ref[...]
    f = 1.0 / (1.0 + jnp.exp(-z))
    cat = jnp.concatenate([f[0:1, :], f[1:2, :]], axis=1)
    d1 = jnp.dot(cat, Wm1_ref[...].T, preferred_element_type=jnp.float32) + bm1_ref[...]
    d2 = jnp.sum(d1 * Wm2_ref[...], axis=1, keepdims=True) + bm2_ref[...]
    out_ref[...] = 1.0 / (1.0 + jnp.exp(-d2))


def _tc_call(sc_out, W1, b1, W2, b2, W3, b3, Wr, br, Wm1, bm1, Wm2, bm2):
    return pl.pallas_call(
        _tc_body,
        out_shape=jax.ShapeDtypeStruct((1, 1), jnp.float32),
    )(sc_out, W1, b1, W2, b2, W3, b3, Wr, br, Wm1, bm1, Wm2, bm2)


def kernel(feat_p, feat_s, edge_index_p, edge_index_s,
           W1, b1, W2, b2, W3, b3, Wr, br, Wm1, bm1, Wm2, bm2):
    sc_out = _sc_call(edge_index_p, edge_index_s, feat_p, feat_s)
    out = _tc_call(sc_out,
                   W1, b1.reshape(1, D), W2, b2.reshape(1, D),
                   W3, b3.reshape(1, D), Wr, br.reshape(1, D),
                   Wm1, bm1.reshape(1, D), Wm2, bm2.reshape(1, 1))
    return out.reshape(1)
